# baseline (speedup 1.0000x reference)
import functools

import jax
import jax.numpy as jnp
from jax import lax
from jax.experimental import pallas as pl
from jax.experimental.pallas import tpu as pltpu
from jax.experimental.pallas import tpu_sc as plsc

N = 10000
NP = 10240
E = 320000
EP = 327680
D = 128
NG = 8
NC = 2
NS = 16
NW = NC * NS
EPW = EP // NW
CHUNK = 128
ROWS_PER_TILE = NP // NS

_BI = 256
_BJ = 512


def _sc_mesh():
    return plsc.VectorSubcoreMesh(core_axis_name="c", subcore_axis_name="s",
                                  num_cores=NC, num_subcores=NS)


def _sc_deg(dste, ones_c, zeros_c):
    @functools.partial(
        pl.kernel,
        out_type=jax.ShapeDtypeStruct((NC, NP, D), jnp.float32),
        mesh=_sc_mesh(),
        scratch_types=[
            pltpu.VMEM((CHUNK,), jnp.int32),
            pltpu.VMEM((CHUNK, D), jnp.float32),
            pltpu.VMEM((64, D), jnp.float32),
            pltpu.VMEM_SHARED((NP, D), jnp.float32),
            pltpu.SemaphoreType.DMA,
        ],
    )
    def deg_kernel(dst_hbm, ones_hbm, z_hbm, out_hbm, idx_v, ones_v, zb_v,
                   acc_sh, sem):
        cid = lax.axis_index("c")
        sid = lax.axis_index("s")
        w = cid * NS + sid

        pltpu.sync_copy(ones_hbm, ones_v)
        pltpu.sync_copy(z_hbm, zb_v)

        def zcp(t, carry):
            pltpu.sync_copy(zb_v, acc_sh.at[pl.ds(sid * ROWS_PER_TILE + t * 64, 64)])
            return carry
        lax.fori_loop(0, ROWS_PER_TILE // 64, zcp, 0)
        plsc.subcore_barrier()

        def step(cix, carry):
            base = w * EPW + cix * CHUNK
            pltpu.sync_copy(dst_hbm.at[pl.ds(base, CHUNK)], idx_v)
            pltpu.sync_copy(ones_v, acc_sh.at[idx_v], add=True)
            return carry
        lax.fori_loop(0, EPW // CHUNK, step, 0)
        plsc.subcore_barrier()

        pltpu.sync_copy(acc_sh.at[pl.ds(sid * ROWS_PER_TILE, ROWS_PER_TILE)],
                        out_hbm.at[cid, pl.ds(sid * ROWS_PER_TILE, ROWS_PER_TILE)])

    return deg_kernel(dste, ones_c, zeros_c)


_K0 = 28
_K1 = 132
_E0 = _K0 * CHUNK * NS
_NBUF = 2


def _sc_agg(y, srce, dste, zeros_d):
    @functools.partial(
        pl.kernel,
        out_type=jax.ShapeDtypeStruct((NC, NP, D), jnp.float32),
        mesh=_sc_mesh(),
        scratch_types=[
            pltpu.VMEM((_NBUF, CHUNK), jnp.int32),
            pltpu.VMEM((_NBUF, CHUNK), jnp.int32),
            pltpu.VMEM((_NBUF, CHUNK, D), jnp.float32),
            pltpu.VMEM((64, D), jnp.float32),
            pltpu.VMEM_SHARED((NP, D), jnp.float32),
            pltpu.SemaphoreType.DMA,
        ],
    )
    def agg_kernel(y_hbm, src_hbm, dst_hbm, z_hbm, out_hbm,
                   sidx_v, didx_v, rows_v, zb_v, acc_sh, sem):
        cid = lax.axis_index("c")
        sid = lax.axis_index("s")

        pltpu.sync_copy(z_hbm, zb_v)

        def zcp(t, carry):
            pltpu.sync_copy(zb_v, acc_sh.at[pl.ds(sid * ROWS_PER_TILE + t * 64, 64)])
            return carry
        lax.fori_loop(0, ROWS_PER_TILE // 64, zcp, 0)
        plsc.subcore_barrier()

        nch = jnp.where(cid == 0, _K0, _K1)
        tstart = jnp.where(cid == 0, sid * (_K0 * CHUNK),
                           _E0 + sid * (_K1 * CHUNK))

        lag = _NBUF - 1

        def step(cix, carry):
            par = lax.rem(cix, _NBUF)

            @pl.when(cix < nch)
            def _():
                base = tstart + cix * CHUNK
                pltpu.sync_copy(src_hbm.at[pl.ds(base, CHUNK)], sidx_v.at[par])
                pltpu.sync_copy(dst_hbm.at[pl.ds(base, CHUNK)], didx_v.at[par])
                pltpu.async_copy(y_hbm.at[sidx_v.at[par]], rows_v.at[par], sem)

            @pl.when(cix >= lag)
            def _():
                prv = lax.rem(cix + 1, _NBUF)
                pltpu.make_async_copy(y_hbm.at[sidx_v.at[prv]],
                                      rows_v.at[prv], sem).wait()
                pltpu.sync_copy(rows_v.at[prv], acc_sh.at[didx_v.at[prv]],
                                add=True)
            return carry
        lax.fori_loop(0, nch + lag, step, 0)
        plsc.subcore_barrier()

        pltpu.sync_copy(acc_sh.at[pl.ds(sid * ROWS_PER_TILE, ROWS_PER_TILE)],
                        out_hbm.at[cid, pl.ds(sid * ROWS_PER_TILE, ROWS_PER_TILE)])

    return agg_kernel(y, srce, dste, zeros_d)


_BM = 256


def _mm(xp, W):
    def body(x_ref, w_ref, o_ref):
        o_ref[...] = jnp.dot(x_ref[...], w_ref[...],
                             preferred_element_type=jnp.float32)
    return pl.pallas_call(
        body,
        grid=(NP // _BM,),
        in_specs=[pl.BlockSpec((_BM, D), lambda i: (i, 0)),
                  pl.BlockSpec((D, D), lambda i: (0, 0))],
        out_specs=pl.BlockSpec((_BM, D), lambda i: (i, 0)),
        out_shape=jax.ShapeDtypeStruct((NP, D), jnp.float32),
    )(xp, W)


def _scale(xw, degp):
    def body(xw_ref, dg_ref, y_ref, dv_ref):
        deg = dg_ref[0, :, 0:1] + dg_ref[1, :, 0:1] + 1.0
        dv = lax.rsqrt(deg)
        y_ref[...] = xw_ref[...] * dv
        dv_ref[...] = dv
    return pl.pallas_call(
        body,
        grid=(NP // _BM,),
        in_specs=[pl.BlockSpec((_BM, D), lambda i: (i, 0)),
                  pl.BlockSpec((NC, _BM, D), lambda i: (0, i, 0))],
        out_specs=[pl.BlockSpec((_BM, D), lambda i: (i, 0)),
                   pl.BlockSpec((_BM, 1), lambda i: (i, 0))],
        out_shape=[jax.ShapeDtypeStruct((NP, D), jnp.float32),
                   jax.ShapeDtypeStruct((NP, 1), jnp.float32)],
    )(xw, degp)


def _layer(aggp, y, dinv, b, W2):
    def body(ag_ref, y_ref, dv_ref, b_ref, w_ref, o_ref):
        dv = dv_ref[...]
        h = jnp.maximum(dv * (ag_ref[0] + ag_ref[1] + y_ref[...]) + b_ref[...],
                        0.0)
        o_ref[...] = jnp.dot(h, w_ref[...],
                             preferred_element_type=jnp.float32) * dv
    return pl.pallas_call(
        body,
        grid=(NP // _BM,),
        in_specs=[pl.BlockSpec((NC, _BM, D), lambda i: (0, i, 0)),
                  pl.BlockSpec((_BM, D), lambda i: (i, 0)),
                  pl.BlockSpec((_BM, 1), lambda i: (i, 0)),
                  pl.BlockSpec((1, D), lambda i: (0, 0)),
                  pl.BlockSpec((D, D), lambda i: (0, 0))],
        out_specs=pl.BlockSpec((_BM, D), lambda i: (i, 0)),
        out_shape=jax.ShapeDtypeStruct((NP, D), jnp.float32),
    )(aggp, y, dinv, b, W2)


def _score(aggp, y, dinv, b, prow):
    def body(ag_ref, y_ref, dv_ref, b_ref, p_ref, val_ref, s_ref):
        dv = dv_ref[...]
        h = jnp.maximum(dv * (ag_ref[0] + ag_ref[1] + y_ref[...]) + b_ref[...],
                        0.0)
        pv = p_ref[...]
        pn = jnp.sqrt(jnp.sum(pv * pv))
        s = jnp.tanh(jnp.sum(h * pv, axis=1, keepdims=True) / pn)
        val_ref[...] = h * s
        s_ref[...] = s
    return pl.pallas_call(
        body,
        grid=(NP // _BM,),
        in_specs=[pl.BlockSpec((NC, _BM, D), lambda i: (0, i, 0)),
                  pl.BlockSpec((_BM, D), lambda i: (i, 0)),
                  pl.BlockSpec((_BM, 1), lambda i: (i, 0)),
                  pl.BlockSpec((1, D), lambda i: (0, 0)),
                  pl.BlockSpec((1, D), lambda i: (0, 0))],
        out_specs=[pl.BlockSpec((_BM, D), lambda i: (i, 0)),
                   pl.BlockSpec((_BM, 1), lambda i: (i, 0))],
        out_shape=[jax.ShapeDtypeStruct((NP, D), jnp.float32),
                   jax.ShapeDtypeStruct((NP, 1), jnp.float32)],
    )(aggp, y, dinv, b, prow)


def _counts(brow):
    def body(b_ref, o_ref):
        j = pl.program_id(0)

        @pl.when(j == 0)
        def _():
            o_ref[...] = jnp.zeros_like(o_ref)

        g = lax.broadcasted_iota(jnp.int32, (NG, _BJ), 0)
        eq = (b_ref[...] == g)
        o_ref[...] += jnp.sum(eq.astype(jnp.float32), axis=1, keepdims=True)
    return pl.pallas_call(
        body,
        grid=(NP // _BJ,),
        in_specs=[pl.BlockSpec((1, _BJ), lambda j: (0, j))],
        out_specs=pl.BlockSpec((NG, 1), lambda j: (0, 0)),
        out_shape=jax.ShapeDtypeStruct((NG, 1), jnp.float32),
    )(brow)


def _rank(scol, srow, bcol, brow):
    nbj = NP // _BJ

    def body(sc_ref, sr_ref, bc_ref, br_ref, o_ref):
        i = pl.program_id(0)
        bc = bc_ref[...]
        sc = sc_ref[...]
        bc_min = bc_ref[0, 0]
        bc_max = bc_ref[_BI - 1, 0]
        ii = i * _BI + lax.broadcasted_iota(jnp.int32, (_BI, _BJ), 0)
        jt = lax.broadcasted_iota(jnp.int32, (_BI, _BJ), 1)
        o_ref[...] = jnp.zeros_like(o_ref)
        for jj in range(nbj):
            br_min = br_ref[0, jj * _BJ]
            br_max = br_ref[0, jj * _BJ + _BJ - 1]
            overlap = (bc_max >= br_min) & (bc_min <= br_max)

            @pl.when(overlap)
            def _(jj=jj):
                sr = sr_ref[0:1, jj * _BJ:(jj + 1) * _BJ]
                br = br_ref[0:1, jj * _BJ:(jj + 1) * _BJ]
                before = (sr > sc) | ((sr == sc) & (jt + jj * _BJ < ii))
                m = (br == bc) & before
                o_ref[...] += jnp.sum(m.astype(jnp.float32), axis=1,
                                      keepdims=True)

    return pl.pallas_call(
        body,
        grid=(NP // _BI,),
        in_specs=[pl.BlockSpec((_BI, 1), lambda i: (i, 0)),
                  pl.BlockSpec((1, NP), lambda i: (0, 0)),
                  pl.BlockSpec((_BI, 1), lambda i: (i, 0)),
                  pl.BlockSpec((1, NP), lambda i: (0, 0))],
        out_specs=pl.BlockSpec((_BI, 1), lambda i: (i, 0)),
        out_shape=jax.ShapeDtypeStruct((NP, 1), jnp.float32),
    )(scol, srow, bcol, brow)


def _pool(val, brow, rrow, ncnt, Wl, bl):
    nblk = NP // _BM

    def body(v_ref, b_ref, r_ref, n_ref, wl_ref, bl_ref, o_ref, acc_ref):
        i = pl.program_id(0)

        @pl.when(i == 0)
        def _():
            acc_ref[...] = jnp.zeros_like(acc_ref)

        km = jnp.ceil(0.5 * n_ref[...])
        g = lax.broadcasted_iota(jnp.int32, (NG, _BM), 0)
        sel = (b_ref[...] == g) & (r_ref[...] < km)
        M = sel.astype(jnp.float32)
        acc_ref[...] += jnp.dot(M, v_ref[...],
                                preferred_element_type=jnp.float32)

        @pl.when(i == nblk - 1)
        def _():
            pooled = acc_ref[...] / jnp.maximum(km, 1.0)
            logits = jnp.dot(pooled, wl_ref[...],
                             preferred_element_type=jnp.float32) + bl_ref[...]
            mx = jnp.max(logits, axis=1, keepdims=True)
            lse = jnp.log(jnp.sum(jnp.exp(logits - mx), axis=1,
                                  keepdims=True)) + mx
            o_ref[...] = logits - lse

    return pl.pallas_call(
        body,
        grid=(nblk,),
        in_specs=[pl.BlockSpec((_BM, D), lambda i: (i, 0)),
                  pl.BlockSpec((1, _BM), lambda i: (0, i)),
                  pl.BlockSpec((1, _BM), lambda i: (0, i)),
                  pl.BlockSpec((NG, 1), lambda i: (0, 0)),
                  pl.BlockSpec((D, 10), lambda i: (0, 0)),
                  pl.BlockSpec((1, 10), lambda i: (0, 0))],
        out_specs=pl.BlockSpec((NG, 10), lambda i: (0, 0)),
        out_shape=jax.ShapeDtypeStruct((NG, 10), jnp.float32),
        scratch_shapes=[pltpu.VMEM((NG, D), jnp.float32)],
    )(val, brow, rrow, ncnt, Wl, bl)


def kernel(x, edge_index, batch, W1, b1, W2, b2, p, Wl, bl):
    src = edge_index[0]
    dst = edge_index[1]
    epad = jnp.full((EP - E,), NP - 1, dtype=jnp.int32)
    srce = jnp.concatenate([src, epad])
    dste = jnp.concatenate([dst, epad])
    xp = jnp.concatenate([x, jnp.zeros((NP - N, D), jnp.float32)], axis=0)
    batchp = jnp.concatenate([batch, jnp.full((NP - N,), NG, jnp.int32)])
    bcol = batchp.reshape(NP, 1)
    brow = batchp.reshape(1, NP)

    ones_c = jnp.ones((CHUNK, D), jnp.float32)
    zeros_d = jnp.zeros((64, D), jnp.float32)

    degp = _sc_deg(dste, ones_c, zeros_d)
    xw1 = _mm(xp, W1)
    y1, dinv = _scale(xw1, degp)
    aggp1 = _sc_agg(y1, srce, dste, zeros_d)
    y2 = _layer(aggp1, y1, dinv, b1.reshape(1, D), W2)
    aggp2 = _sc_agg(y2, srce, dste, zeros_d)
    val, s = _score(aggp2, y2, dinv, b2.reshape(1, D), p.reshape(1, D))
    ncnt = _counts(brow)
    rank = _rank(s, s.reshape(1, NP), bcol, brow)
    out = _pool(val, brow, rank.reshape(1, NP), ncnt, Wl, bl.reshape(1, 10))
    return out


# speedup vs baseline: 1.1802x; 1.1802x over previous
import functools

import jax
import jax.numpy as jnp
from jax import lax
from jax.experimental import pallas as pl
from jax.experimental.pallas import tpu as pltpu
from jax.experimental.pallas import tpu_sc as plsc

N = 10000
NP = 10240
E = 320000
EP = 327680
D = 128
NG = 8
NC = 2
NS = 16
NW = NC * NS
EPW = EP // NW
CHUNK = 128
ROWS_PER_TILE = NP // NS

_BI = 256
_BJ = 512


def _sc_mesh():
    return plsc.VectorSubcoreMesh(core_axis_name="c", subcore_axis_name="s",
                                  num_cores=NC, num_subcores=NS)


def _sc_deg(dste, ones_c, zeros_c):
    @functools.partial(
        pl.kernel,
        out_type=jax.ShapeDtypeStruct((NC, NP, D), jnp.float32),
        mesh=_sc_mesh(),
        scratch_types=[
            pltpu.VMEM((CHUNK,), jnp.int32),
            pltpu.VMEM((CHUNK, D), jnp.float32),
            pltpu.VMEM((64, D), jnp.float32),
            pltpu.VMEM_SHARED((NP, D), jnp.float32),
            pltpu.SemaphoreType.DMA,
        ],
    )
    def deg_kernel(dst_hbm, ones_hbm, z_hbm, out_hbm, idx_v, ones_v, zb_v,
                   acc_sh, sem):
        cid = lax.axis_index("c")
        sid = lax.axis_index("s")
        w = cid * NS + sid

        pltpu.sync_copy(ones_hbm, ones_v)
        pltpu.sync_copy(z_hbm, zb_v)

        def zcp(t, carry):
            pltpu.sync_copy(zb_v, acc_sh.at[pl.ds(sid * ROWS_PER_TILE + t * 64, 64)])
            return carry
        lax.fori_loop(0, ROWS_PER_TILE // 64, zcp, 0)
        plsc.subcore_barrier()

        def step(cix, carry):
            base = w * EPW + cix * CHUNK
            pltpu.sync_copy(dst_hbm.at[pl.ds(base, CHUNK)], idx_v)
            pltpu.sync_copy(ones_v, acc_sh.at[idx_v], add=True)
            return carry
        lax.fori_loop(0, EPW // CHUNK, step, 0)
        plsc.subcore_barrier()

        pltpu.sync_copy(acc_sh.at[pl.ds(sid * ROWS_PER_TILE, ROWS_PER_TILE)],
                        out_hbm.at[cid, pl.ds(sid * ROWS_PER_TILE, ROWS_PER_TILE)])

    return deg_kernel(dste, ones_c, zeros_c)


_K0 = 132
_K1 = 28
_E0 = _K0 * CHUNK * NS
_NBUF = 2


def _sc_agg(y, srce, dste, zeros_d):
    @functools.partial(
        pl.kernel,
        out_type=jax.ShapeDtypeStruct((NC, NP, D), jnp.float32),
        mesh=_sc_mesh(),
        scratch_types=[
            pltpu.VMEM((_NBUF, CHUNK), jnp.int32),
            pltpu.VMEM((_NBUF, CHUNK), jnp.int32),
            pltpu.VMEM((_NBUF, CHUNK, D), jnp.float32),
            pltpu.VMEM((64, D), jnp.float32),
            pltpu.VMEM_SHARED((NP, D), jnp.float32),
            pltpu.SemaphoreType.DMA,
        ],
    )
    def agg_kernel(y_hbm, src_hbm, dst_hbm, z_hbm, out_hbm,
                   sidx_v, didx_v, rows_v, zb_v, acc_sh, sem):
        cid = lax.axis_index("c")
        sid = lax.axis_index("s")

        pltpu.sync_copy(z_hbm, zb_v)

        def zcp(t, carry):
            pltpu.sync_copy(zb_v, acc_sh.at[pl.ds(sid * ROWS_PER_TILE + t * 64, 64)])
            return carry
        lax.fori_loop(0, ROWS_PER_TILE // 64, zcp, 0)
        plsc.subcore_barrier()

        nch = jnp.where(cid == 0, _K0, _K1)
        tstart = jnp.where(cid == 0, sid * (_K0 * CHUNK),
                           _E0 + sid * (_K1 * CHUNK))

        lag = _NBUF - 1

        def step(cix, carry):
            par = lax.rem(cix, _NBUF)

            @pl.when(cix < nch)
            def _():
                base = tstart + cix * CHUNK
                pltpu.sync_copy(src_hbm.at[pl.ds(base, CHUNK)], sidx_v.at[par])
                pltpu.sync_copy(dst_hbm.at[pl.ds(base, CHUNK)], didx_v.at[par])
                pltpu.async_copy(y_hbm.at[sidx_v.at[par]], rows_v.at[par], sem)

            @pl.when(cix >= lag)
            def _():
                prv = lax.rem(cix + 1, _NBUF)
                pltpu.make_async_copy(y_hbm.at[sidx_v.at[prv]],
                                      rows_v.at[prv], sem).wait()
                pltpu.sync_copy(rows_v.at[prv], acc_sh.at[didx_v.at[prv]],
                                add=True)
            return carry
        lax.fori_loop(0, nch + lag, step, 0)
        plsc.subcore_barrier()

        pltpu.sync_copy(acc_sh.at[pl.ds(sid * ROWS_PER_TILE, ROWS_PER_TILE)],
                        out_hbm.at[cid, pl.ds(sid * ROWS_PER_TILE, ROWS_PER_TILE)])

    return agg_kernel(y, srce, dste, zeros_d)


_BM = 256


def _mm(xp, W):
    def body(x_ref, w_ref, o_ref):
        o_ref[...] = jnp.dot(x_ref[...], w_ref[...],
                             preferred_element_type=jnp.float32)
    return pl.pallas_call(
        body,
        grid=(NP // _BM,),
        in_specs=[pl.BlockSpec((_BM, D), lambda i: (i, 0)),
                  pl.BlockSpec((D, D), lambda i: (0, 0))],
        out_specs=pl.BlockSpec((_BM, D), lambda i: (i, 0)),
        out_shape=jax.ShapeDtypeStruct((NP, D), jnp.float32),
    )(xp, W)


def _scale(xw, degp):
    def body(xw_ref, dg_ref, y_ref, dv_ref):
        deg = dg_ref[0, :, 0:1] + dg_ref[1, :, 0:1] + 1.0
        dv = lax.rsqrt(deg)
        y_ref[...] = xw_ref[...] * dv
        dv_ref[...] = dv
    return pl.pallas_call(
        body,
        grid=(NP // _BM,),
        in_specs=[pl.BlockSpec((_BM, D), lambda i: (i, 0)),
                  pl.BlockSpec((NC, _BM, D), lambda i: (0, i, 0))],
        out_specs=[pl.BlockSpec((_BM, D), lambda i: (i, 0)),
                   pl.BlockSpec((_BM, 1), lambda i: (i, 0))],
        out_shape=[jax.ShapeDtypeStruct((NP, D), jnp.float32),
                   jax.ShapeDtypeStruct((NP, 1), jnp.float32)],
    )(xw, degp)


def _layer(aggp, y, dinv, b, W2):
    def body(ag_ref, y_ref, dv_ref, b_ref, w_ref, o_ref):
        dv = dv_ref[...]
        h = jnp.maximum(dv * (ag_ref[0] + ag_ref[1] + y_ref[...]) + b_ref[...],
                        0.0)
        o_ref[...] = jnp.dot(h, w_ref[...],
                             preferred_element_type=jnp.float32) * dv
    return pl.pallas_call(
        body,
        grid=(NP // _BM,),
        in_specs=[pl.BlockSpec((NC, _BM, D), lambda i: (0, i, 0)),
                  pl.BlockSpec((_BM, D), lambda i: (i, 0)),
                  pl.BlockSpec((_BM, 1), lambda i: (i, 0)),
                  pl.BlockSpec((1, D), lambda i: (0, 0)),
                  pl.BlockSpec((D, D), lambda i: (0, 0))],
        out_specs=pl.BlockSpec((_BM, D), lambda i: (i, 0)),
        out_shape=jax.ShapeDtypeStruct((NP, D), jnp.float32),
    )(aggp, y, dinv, b, W2)


def _score(aggp, y, dinv, b, prow):
    def body(ag_ref, y_ref, dv_ref, b_ref, p_ref, val_ref, s_ref):
        dv = dv_ref[...]
        h = jnp.maximum(dv * (ag_ref[0] + ag_ref[1] + y_ref[...]) + b_ref[...],
                        0.0)
        pv = p_ref[...]
        pn = jnp.sqrt(jnp.sum(pv * pv))
        s = jnp.tanh(jnp.sum(h * pv, axis=1, keepdims=True) / pn)
        val_ref[...] = h * s
        s_ref[...] = s
    return pl.pallas_call(
        body,
        grid=(NP // _BM,),
        in_specs=[pl.BlockSpec((NC, _BM, D), lambda i: (0, i, 0)),
                  pl.BlockSpec((_BM, D), lambda i: (i, 0)),
                  pl.BlockSpec((_BM, 1), lambda i: (i, 0)),
                  pl.BlockSpec((1, D), lambda i: (0, 0)),
                  pl.BlockSpec((1, D), lambda i: (0, 0))],
        out_specs=[pl.BlockSpec((_BM, D), lambda i: (i, 0)),
                   pl.BlockSpec((_BM, 1), lambda i: (i, 0))],
        out_shape=[jax.ShapeDtypeStruct((NP, D), jnp.float32),
                   jax.ShapeDtypeStruct((NP, 1), jnp.float32)],
    )(aggp, y, dinv, b, prow)


def _counts(brow):
    def body(b_ref, o_ref):
        j = pl.program_id(0)

        @pl.when(j == 0)
        def _():
            o_ref[...] = jnp.zeros_like(o_ref)

        g = lax.broadcasted_iota(jnp.int32, (NG, _BJ), 0)
        eq = (b_ref[...] == g)
        o_ref[...] += jnp.sum(eq.astype(jnp.float32), axis=1, keepdims=True)
    return pl.pallas_call(
        body,
        grid=(NP // _BJ,),
        in_specs=[pl.BlockSpec((1, _BJ), lambda j: (0, j))],
        out_specs=pl.BlockSpec((NG, 1), lambda j: (0, 0)),
        out_shape=jax.ShapeDtypeStruct((NG, 1), jnp.float32),
    )(brow)


def _rank(scol, srow, bcol, brow):
    nbj = NP // _BJ

    def body(sc_ref, sr_ref, bc_ref, br_ref, o_ref):
        i = pl.program_id(0)
        bc = bc_ref[...]
        sc = sc_ref[...]
        bc_min = bc_ref[0, 0]
        bc_max = bc_ref[_BI - 1, 0]
        o_ref[...] = jnp.zeros_like(o_ref)
        for jj in range(nbj):
            br_min = br_ref[0, jj * _BJ]
            br_max = br_ref[0, jj * _BJ + _BJ - 1]
            overlap = (bc_max >= br_min) & (bc_min <= br_max)

            left = (jj + 1) * _BJ - 1 < i * _BI
            right = jj * _BJ > i * _BI + _BI - 1
            sl = slice(jj * _BJ, (jj + 1) * _BJ)

            @pl.when(overlap & left)
            def _(sl=sl):
                m = (br_ref[0:1, sl] == bc) & (sr_ref[0:1, sl] >= sc)
                o_ref[...] += jnp.sum(m.astype(jnp.float32), axis=1,
                                      keepdims=True)

            @pl.when(overlap & right)
            def _(sl=sl):
                m = (br_ref[0:1, sl] == bc) & (sr_ref[0:1, sl] > sc)
                o_ref[...] += jnp.sum(m.astype(jnp.float32), axis=1,
                                      keepdims=True)

            @pl.when(overlap & jnp.logical_not(left | right))
            def _(sl=sl, jj=jj):
                sr = sr_ref[0:1, sl]
                ii = i * _BI + lax.broadcasted_iota(jnp.int32, (_BI, _BJ), 0)
                jt = jj * _BJ + lax.broadcasted_iota(jnp.int32, (_BI, _BJ), 1)
                before = (sr > sc) | ((sr == sc) & (jt < ii))
                m = (br_ref[0:1, sl] == bc) & before
                o_ref[...] += jnp.sum(m.astype(jnp.float32), axis=1,
                                      keepdims=True)

    return pl.pallas_call(
        body,
        grid=(NP // _BI,),
        in_specs=[pl.BlockSpec((_BI, 1), lambda i: (i, 0)),
                  pl.BlockSpec((1, NP), lambda i: (0, 0)),
                  pl.BlockSpec((_BI, 1), lambda i: (i, 0)),
                  pl.BlockSpec((1, NP), lambda i: (0, 0))],
        out_specs=pl.BlockSpec((_BI, 1), lambda i: (i, 0)),
        out_shape=jax.ShapeDtypeStruct((NP, 1), jnp.float32),
    )(scol, srow, bcol, brow)


def _pool(val, brow, rrow, ncnt, Wl, bl):
    nblk = NP // _BM

    def body(v_ref, b_ref, r_ref, n_ref, wl_ref, bl_ref, o_ref, acc_ref):
        i = pl.program_id(0)

        @pl.when(i == 0)
        def _():
            acc_ref[...] = jnp.zeros_like(acc_ref)

        km = jnp.ceil(0.5 * n_ref[...])
        g = lax.broadcasted_iota(jnp.int32, (NG, _BM), 0)
        sel = (b_ref[...] == g) & (r_ref[...] < km)
        M = sel.astype(jnp.float32)
        acc_ref[...] += jnp.dot(M, v_ref[...],
                                preferred_element_type=jnp.float32)

        @pl.when(i == nblk - 1)
        def _():
            pooled = acc_ref[...] / jnp.maximum(km, 1.0)
            logits = jnp.dot(pooled, wl_ref[...],
                             preferred_element_type=jnp.float32) + bl_ref[...]
            mx = jnp.max(logits, axis=1, keepdims=True)
            lse = jnp.log(jnp.sum(jnp.exp(logits - mx), axis=1,
                                  keepdims=True)) + mx
            o_ref[...] = logits - lse

    return pl.pallas_call(
        body,
        grid=(nblk,),
        in_specs=[pl.BlockSpec((_BM, D), lambda i: (i, 0)),
                  pl.BlockSpec((1, _BM), lambda i: (0, i)),
                  pl.BlockSpec((1, _BM), lambda i: (0, i)),
                  pl.BlockSpec((NG, 1), lambda i: (0, 0)),
                  pl.BlockSpec((D, 10), lambda i: (0, 0)),
                  pl.BlockSpec((1, 10), lambda i: (0, 0))],
        out_specs=pl.BlockSpec((NG, 10), lambda i: (0, 0)),
        out_shape=jax.ShapeDtypeStruct((NG, 10), jnp.float32),
        scratch_shapes=[pltpu.VMEM((NG, D), jnp.float32)],
    )(val, brow, rrow, ncnt, Wl, bl)


def kernel(x, edge_index, batch, W1, b1, W2, b2, p, Wl, bl):
    src = edge_index[0]
    dst = edge_index[1]
    epad = jnp.full((EP - E,), NP - 1, dtype=jnp.int32)
    srce = jnp.concatenate([src, epad])
    dste = jnp.concatenate([dst, epad])
    xp = jnp.concatenate([x, jnp.zeros((NP - N, D), jnp.float32)], axis=0)
    batchp = jnp.concatenate([batch, jnp.full((NP - N,), NG, jnp.int32)])
    bcol = batchp.reshape(NP, 1)
    brow = batchp.reshape(1, NP)

    ones_c = jnp.ones((CHUNK, D), jnp.float32)
    zeros_d = jnp.zeros((64, D), jnp.float32)

    degp = _sc_deg(dste, ones_c, zeros_d)
    xw1 = _mm(xp, W1)
    y1, dinv = _scale(xw1, degp)
    aggp1 = _sc_agg(y1, srce, dste, zeros_d)
    y2 = _layer(aggp1, y1, dinv, b1.reshape(1, D), W2)
    aggp2 = _sc_agg(y2, srce, dste, zeros_d)
    val, s = _score(aggp2, y2, dinv, b2.reshape(1, D), p.reshape(1, D))
    ncnt = _counts(brow)
    rank = _rank(s, s.reshape(1, NP), bcol, brow)
    out = _pool(val, brow, rank.reshape(1, NP), ncnt, Wl, bl.reshape(1, 10))
    return out


# speedup vs baseline: 1.2441x; 1.0542x over previous
import functools

import jax
import jax.numpy as jnp
from jax import lax
from jax.experimental import pallas as pl
from jax.experimental.pallas import tpu as pltpu
from jax.experimental.pallas import tpu_sc as plsc

N = 10000
NP = 10240
E = 320000
EP = 327680
D = 128
NG = 8
NC = 2
NS = 16
NW = NC * NS
EPW = EP // NW
CHUNK = 128
ROWS_PER_TILE = NP // NS

_BI = 512
_BJ = 1024


def _sc_mesh():
    return plsc.VectorSubcoreMesh(core_axis_name="c", subcore_axis_name="s",
                                  num_cores=NC, num_subcores=NS)


def _sc_deg(dste, ones_c, zeros_c):
    @functools.partial(
        pl.kernel,
        out_type=jax.ShapeDtypeStruct((NC, NP, D), jnp.float32),
        mesh=_sc_mesh(),
        scratch_types=[
            pltpu.VMEM((CHUNK,), jnp.int32),
            pltpu.VMEM((CHUNK, D), jnp.float32),
            pltpu.VMEM((64, D), jnp.float32),
            pltpu.VMEM_SHARED((NP, D), jnp.float32),
            pltpu.SemaphoreType.DMA,
        ],
    )
    def deg_kernel(dst_hbm, ones_hbm, z_hbm, out_hbm, idx_v, ones_v, zb_v,
                   acc_sh, sem):
        cid = lax.axis_index("c")
        sid = lax.axis_index("s")
        w = cid * NS + sid

        pltpu.sync_copy(ones_hbm, ones_v)
        pltpu.sync_copy(z_hbm, zb_v)

        def zcp(t, carry):
            pltpu.sync_copy(zb_v, acc_sh.at[pl.ds(sid * ROWS_PER_TILE + t * 64, 64)])
            return carry
        lax.fori_loop(0, ROWS_PER_TILE // 64, zcp, 0)
        plsc.subcore_barrier()

        def step(cix, carry):
            base = w * EPW + cix * CHUNK
            pltpu.sync_copy(dst_hbm.at[pl.ds(base, CHUNK)], idx_v)
            pltpu.sync_copy(ones_v, acc_sh.at[idx_v], add=True)
            return carry
        lax.fori_loop(0, EPW // CHUNK, step, 0)
        plsc.subcore_barrier()

        pltpu.sync_copy(acc_sh.at[pl.ds(sid * ROWS_PER_TILE, ROWS_PER_TILE)],
                        out_hbm.at[cid, pl.ds(sid * ROWS_PER_TILE, ROWS_PER_TILE)])

    return deg_kernel(dste, ones_c, zeros_c)


_K0 = 117
_K1 = 43
_E0 = _K0 * CHUNK * NS
_NBUF = 2


def _sc_agg(y, srce, dste, zeros_d):
    @functools.partial(
        pl.kernel,
        out_type=jax.ShapeDtypeStruct((NC, NP, D), jnp.float32),
        mesh=_sc_mesh(),
        scratch_types=[
            pltpu.VMEM((_NBUF, CHUNK), jnp.int32),
            pltpu.VMEM((_NBUF, CHUNK), jnp.int32),
            pltpu.VMEM((_NBUF, CHUNK, D), jnp.float32),
            pltpu.VMEM((64, D), jnp.float32),
            pltpu.VMEM_SHARED((NP, D), jnp.float32),
            pltpu.SemaphoreType.DMA,
        ],
    )
    def agg_kernel(y_hbm, src_hbm, dst_hbm, z_hbm, out_hbm,
                   sidx_v, didx_v, rows_v, zb_v, acc_sh, sem):
        cid = lax.axis_index("c")
        sid = lax.axis_index("s")

        pltpu.sync_copy(z_hbm, zb_v)

        def zcp(t, carry):
            pltpu.sync_copy(zb_v, acc_sh.at[pl.ds(sid * ROWS_PER_TILE + t * 64, 64)])
            return carry
        lax.fori_loop(0, ROWS_PER_TILE // 64, zcp, 0)
        plsc.subcore_barrier()

        nch = jnp.where(cid == 0, _K0, _K1)
        tstart = jnp.where(cid == 0, sid * (_K0 * CHUNK),
                           _E0 + sid * (_K1 * CHUNK))

        lag = _NBUF - 1

        def step(cix, carry):
            par = lax.rem(cix, _NBUF)

            @pl.when(cix < nch)
            def _():
                base = tstart + cix * CHUNK
                pltpu.sync_copy(src_hbm.at[pl.ds(base, CHUNK)], sidx_v.at[par])
                pltpu.sync_copy(dst_hbm.at[pl.ds(base, CHUNK)], didx_v.at[par])
                pltpu.async_copy(y_hbm.at[sidx_v.at[par]], rows_v.at[par], sem)

            @pl.when(cix >= lag)
            def _():
                prv = lax.rem(cix + 1, _NBUF)
                pltpu.make_async_copy(y_hbm.at[sidx_v.at[prv]],
                                      rows_v.at[prv], sem).wait()
                pltpu.sync_copy(rows_v.at[prv], acc_sh.at[didx_v.at[prv]],
                                add=True)
            return carry
        lax.fori_loop(0, nch + lag, step, 0)
        plsc.subcore_barrier()

        pltpu.sync_copy(acc_sh.at[pl.ds(sid * ROWS_PER_TILE, ROWS_PER_TILE)],
                        out_hbm.at[cid, pl.ds(sid * ROWS_PER_TILE, ROWS_PER_TILE)])

    return agg_kernel(y, srce, dste, zeros_d)


_BM = 256


def _mm(xp, W):
    def body(x_ref, w_ref, o_ref):
        o_ref[...] = jnp.dot(x_ref[...], w_ref[...],
                             preferred_element_type=jnp.float32)
    return pl.pallas_call(
        body,
        grid=(NP // _BM,),
        in_specs=[pl.BlockSpec((_BM, D), lambda i: (i, 0)),
                  pl.BlockSpec((D, D), lambda i: (0, 0))],
        out_specs=pl.BlockSpec((_BM, D), lambda i: (i, 0)),
        out_shape=jax.ShapeDtypeStruct((NP, D), jnp.float32),
    )(xp, W)


def _scale(xw, degp):
    def body(xw_ref, dg_ref, y_ref, dv_ref):
        deg = dg_ref[0, :, 0:1] + dg_ref[1, :, 0:1] + 1.0
        dv = lax.rsqrt(deg)
        y_ref[...] = xw_ref[...] * dv
        dv_ref[...] = dv
    return pl.pallas_call(
        body,
        grid=(NP // _BM,),
        in_specs=[pl.BlockSpec((_BM, D), lambda i: (i, 0)),
                  pl.BlockSpec((NC, _BM, D), lambda i: (0, i, 0))],
        out_specs=[pl.BlockSpec((_BM, D), lambda i: (i, 0)),
                   pl.BlockSpec((_BM, 1), lambda i: (i, 0))],
        out_shape=[jax.ShapeDtypeStruct((NP, D), jnp.float32),
                   jax.ShapeDtypeStruct((NP, 1), jnp.float32)],
    )(xw, degp)


def _layer(aggp, y, dinv, b, W2):
    def body(ag_ref, y_ref, dv_ref, b_ref, w_ref, o_ref):
        dv = dv_ref[...]
        h = jnp.maximum(dv * (ag_ref[0] + ag_ref[1] + y_ref[...]) + b_ref[...],
                        0.0)
        o_ref[...] = jnp.dot(h, w_ref[...],
                             preferred_element_type=jnp.float32) * dv
    return pl.pallas_call(
        body,
        grid=(NP // _BM,),
        in_specs=[pl.BlockSpec((NC, _BM, D), lambda i: (0, i, 0)),
                  pl.BlockSpec((_BM, D), lambda i: (i, 0)),
                  pl.BlockSpec((_BM, 1), lambda i: (i, 0)),
                  pl.BlockSpec((1, D), lambda i: (0, 0)),
                  pl.BlockSpec((D, D), lambda i: (0, 0))],
        out_specs=pl.BlockSpec((_BM, D), lambda i: (i, 0)),
        out_shape=jax.ShapeDtypeStruct((NP, D), jnp.float32),
    )(aggp, y, dinv, b, W2)


def _score(aggp, y, dinv, b, prow):
    def body(ag_ref, y_ref, dv_ref, b_ref, p_ref, val_ref, s_ref):
        dv = dv_ref[...]
        h = jnp.maximum(dv * (ag_ref[0] + ag_ref[1] + y_ref[...]) + b_ref[...],
                        0.0)
        pv = p_ref[...]
        pn = jnp.sqrt(jnp.sum(pv * pv))
        s = jnp.tanh(jnp.sum(h * pv, axis=1, keepdims=True) / pn)
        val_ref[...] = h * s
        s_ref[...] = s
    return pl.pallas_call(
        body,
        grid=(NP // _BM,),
        in_specs=[pl.BlockSpec((NC, _BM, D), lambda i: (0, i, 0)),
                  pl.BlockSpec((_BM, D), lambda i: (i, 0)),
                  pl.BlockSpec((_BM, 1), lambda i: (i, 0)),
                  pl.BlockSpec((1, D), lambda i: (0, 0)),
                  pl.BlockSpec((1, D), lambda i: (0, 0))],
        out_specs=[pl.BlockSpec((_BM, D), lambda i: (i, 0)),
                   pl.BlockSpec((_BM, 1), lambda i: (i, 0))],
        out_shape=[jax.ShapeDtypeStruct((NP, D), jnp.float32),
                   jax.ShapeDtypeStruct((NP, 1), jnp.float32)],
    )(aggp, y, dinv, b, prow)


def _counts(brow):
    def body(b_ref, o_ref):
        j = pl.program_id(0)

        @pl.when(j == 0)
        def _():
            o_ref[...] = jnp.zeros_like(o_ref)

        g = lax.broadcasted_iota(jnp.int32, (NG, _BJ), 0)
        eq = (b_ref[...] == g)
        o_ref[...] += jnp.sum(eq.astype(jnp.float32), axis=1, keepdims=True)
    return pl.pallas_call(
        body,
        grid=(NP // _BJ,),
        in_specs=[pl.BlockSpec((1, _BJ), lambda j: (0, j))],
        out_specs=pl.BlockSpec((NG, 1), lambda j: (0, 0)),
        out_shape=jax.ShapeDtypeStruct((NG, 1), jnp.float32),
    )(brow)


def _rank(scol, srow, bcol, brow):
    nbj = NP // _BJ

    def body(sc_ref, sr_ref, bc_ref, br_ref, o_ref):
        i = pl.program_id(0)
        bc = bc_ref[...]
        sc = sc_ref[...]
        bc_min = bc_ref[0, 0]
        bc_max = bc_ref[_BI - 1, 0]
        o_ref[...] = jnp.zeros_like(o_ref)
        for jj in range(nbj):
            br_min = br_ref[0, jj * _BJ]
            br_max = br_ref[0, jj * _BJ + _BJ - 1]
            overlap = (bc_max >= br_min) & (bc_min <= br_max)

            left = (jj + 1) * _BJ - 1 < i * _BI
            right = jj * _BJ > i * _BI + _BI - 1
            sl = slice(jj * _BJ, (jj + 1) * _BJ)

            @pl.when(overlap & left)
            def _(sl=sl):
                m = (br_ref[0:1, sl] == bc) & (sr_ref[0:1, sl] >= sc)
                o_ref[...] += jnp.sum(m.astype(jnp.float32), axis=1,
                                      keepdims=True)

            @pl.when(overlap & right)
            def _(sl=sl):
                m = (br_ref[0:1, sl] == bc) & (sr_ref[0:1, sl] > sc)
                o_ref[...] += jnp.sum(m.astype(jnp.float32), axis=1,
                                      keepdims=True)

            @pl.when(overlap & jnp.logical_not(left | right))
            def _(sl=sl, jj=jj):
                sr = sr_ref[0:1, sl]
                ii = i * _BI + lax.broadcasted_iota(jnp.int32, (_BI, _BJ), 0)
                jt = jj * _BJ + lax.broadcasted_iota(jnp.int32, (_BI, _BJ), 1)
                before = (sr > sc) | ((sr == sc) & (jt < ii))
                m = (br_ref[0:1, sl] == bc) & before
                o_ref[...] += jnp.sum(m.astype(jnp.float32), axis=1,
                                      keepdims=True)

    return pl.pallas_call(
        body,
        grid=(NP // _BI,),
        in_specs=[pl.BlockSpec((_BI, 1), lambda i: (i, 0)),
                  pl.BlockSpec((1, NP), lambda i: (0, 0)),
                  pl.BlockSpec((_BI, 1), lambda i: (i, 0)),
                  pl.BlockSpec((1, NP), lambda i: (0, 0))],
        out_specs=pl.BlockSpec((_BI, 1), lambda i: (i, 0)),
        out_shape=jax.ShapeDtypeStruct((NP, 1), jnp.float32),
    )(scol, srow, bcol, brow)


def _pool(val, brow, rrow, ncnt, Wl, bl):
    nblk = NP // _BM

    def body(v_ref, b_ref, r_ref, n_ref, wl_ref, bl_ref, o_ref, acc_ref):
        i = pl.program_id(0)

        @pl.when(i == 0)
        def _():
            acc_ref[...] = jnp.zeros_like(acc_ref)

        km = jnp.ceil(0.5 * n_ref[...])
        g = lax.broadcasted_iota(jnp.int32, (NG, _BM), 0)
        sel = (b_ref[...] == g) & (r_ref[...] < km)
        M = sel.astype(jnp.float32)
        acc_ref[...] += jnp.dot(M, v_ref[...],
                                preferred_element_type=jnp.float32)

        @pl.when(i == nblk - 1)
        def _():
            pooled = acc_ref[...] / jnp.maximum(km, 1.0)
            logits = jnp.dot(pooled, wl_ref[...],
                             preferred_element_type=jnp.float32) + bl_ref[...]
            mx = jnp.max(logits, axis=1, keepdims=True)
            lse = jnp.log(jnp.sum(jnp.exp(logits - mx), axis=1,
                                  keepdims=True)) + mx
            o_ref[...] = logits - lse

    return pl.pallas_call(
        body,
        grid=(nblk,),
        in_specs=[pl.BlockSpec((_BM, D), lambda i: (i, 0)),
                  pl.BlockSpec((1, _BM), lambda i: (0, i)),
                  pl.BlockSpec((1, _BM), lambda i: (0, i)),
                  pl.BlockSpec((NG, 1), lambda i: (0, 0)),
                  pl.BlockSpec((D, 10), lambda i: (0, 0)),
                  pl.BlockSpec((1, 10), lambda i: (0, 0))],
        out_specs=pl.BlockSpec((NG, 10), lambda i: (0, 0)),
        out_shape=jax.ShapeDtypeStruct((NG, 10), jnp.float32),
        scratch_shapes=[pltpu.VMEM((NG, D), jnp.float32)],
    )(val, brow, rrow, ncnt, Wl, bl)


def kernel(x, edge_index, batch, W1, b1, W2, b2, p, Wl, bl):
    src = edge_index[0]
    dst = edge_index[1]
    epad = jnp.full((EP - E,), NP - 1, dtype=jnp.int32)
    srce = jnp.concatenate([src, epad])
    dste = jnp.concatenate([dst, epad])
    xp = jnp.concatenate([x, jnp.zeros((NP - N, D), jnp.float32)], axis=0)
    batchp = jnp.concatenate([batch, jnp.full((NP - N,), NG, jnp.int32)])
    bcol = batchp.reshape(NP, 1)
    brow = batchp.reshape(1, NP)

    ones_c = jnp.ones((CHUNK, D), jnp.float32)
    zeros_d = jnp.zeros((64, D), jnp.float32)

    degp = _sc_deg(dste, ones_c, zeros_d)
    xw1 = _mm(xp, W1)
    y1, dinv = _scale(xw1, degp)
    aggp1 = _sc_agg(y1, srce, dste, zeros_d)
    y2 = _layer(aggp1, y1, dinv, b1.reshape(1, D), W2)
    aggp2 = _sc_agg(y2, srce, dste, zeros_d)
    val, s = _score(aggp2, y2, dinv, b2.reshape(1, D), p.reshape(1, D))
    ncnt = _counts(brow)
    rank = _rank(s, s.reshape(1, NP), bcol, brow)
    out = _pool(val, brow, rank.reshape(1, NP), ncnt, Wl, bl.reshape(1, 10))
    return out
